# baseline (speedup 1.0000x reference)
import functools

import jax
import jax.numpy as jnp
from jax import lax
from jax.experimental import pallas as pl
from jax.experimental.pallas import tpu as pltpu
from jax.experimental.pallas import tpu_sc as plsc

NC = 2
NS = 16
NW = NC * NS
L = 16
CHUNK = 128

TILE_COLS = 7813
PLANE = TILE_COLS * 1024


KTILE = 128


def _relayout_body(a, b, oa, ob):
    for src, dst in ((a, oa), (b, ob)):
        x = src[...]
        for t in range(KTILE):
            dst[0, t, :, :] = x[:, t * 128:(t + 1) * 128]


def _relayout(wt, it):
    jblocks = (TILE_COLS + KTILE - 1) // KTILE
    return pl.pallas_call(
        _relayout_body,
        grid=(2, jblocks),
        in_specs=[pl.BlockSpec((8, KTILE * 128), lambda i, j: (i, j)),
                  pl.BlockSpec((8, KTILE * 128), lambda i, j: (i, j))],
        out_specs=[
            pl.BlockSpec((1, KTILE, 8, 128), lambda i, j: (i, j, 0, 0)),
            pl.BlockSpec((1, KTILE, 8, 128), lambda i, j: (i, j, 0, 0))],
        out_shape=[jax.ShapeDtypeStruct((2, TILE_COLS, 8, 128), jnp.float32),
                   jax.ShapeDtypeStruct((2, TILE_COLS, 8, 128), jnp.float32)],
    )(wt, it)


def _mf_sc(B, H):
    assert B % (NW * CHUNK) == 0 and H == L
    bpw = B // NW
    nbc = bpw // CHUNK
    ngather = bpw * H // CHUNK

    mesh = plsc.VectorSubcoreMesh(
        core_axis_name="c", subcore_axis_name="s",
        num_cores=NC, num_subcores=NS)

    @functools.partial(
        pl.kernel,
        mesh=mesh,
        compiler_params=pltpu.CompilerParams(
            needs_layout_passes=False, use_tc_tiling_on_sc=False),
        out_type=[
            jax.ShapeDtypeStruct((B,), jnp.float32),
            jax.ShapeDtypeStruct((NW, L), jnp.float32),
        ],
        scratch_types=[
            pltpu.VMEM((bpw,), jnp.int32),
            pltpu.VMEM((bpw,), jnp.int32),
            pltpu.VMEM((nbc, CHUNK), jnp.int32),
            pltpu.VMEM((nbc, CHUNK), jnp.int32),
            pltpu.VMEM((ngather, CHUNK), jnp.int32),
            pltpu.VMEM((ngather, CHUNK), jnp.int32),
            pltpu.VMEM((ngather, CHUNK), jnp.float32),
            pltpu.VMEM((ngather, CHUNK), jnp.float32),
            pltpu.VMEM((nbc, CHUNK), jnp.float32),
            pltpu.VMEM((nbc, CHUNK), jnp.float32),
            pltpu.VMEM((bpw,), jnp.float32),
            pltpu.VMEM((bpw,), jnp.float32),
            pltpu.VMEM((L,), jnp.float32),
            pltpu.VMEM((1,), jnp.float32),
            pltpu.SemaphoreType.DMA,
            pltpu.SemaphoreType.DMA,
        ],
    )
    def k(user_h, item_h, rating_h, uw_h, iw_h, ub_h, ib_h, bias_h,
          out_h, part_h,
          idx1_u, idx1_i, idxr_u, idxr_i,
          addr_u, addr_i, val_u, val_i, bu, bi,
          rat_v, out_v, loss_v, bias_v, sem0, sem1):
        wid = lax.axis_index("s") * NC + lax.axis_index("c")
        base = wid * bpw
        iota = lax.iota(jnp.int32, L)

        cps = [
            pltpu.async_copy(user_h.at[pl.ds(base, bpw)], idx1_u, sem0),
            pltpu.async_copy(item_h.at[pl.ds(base, bpw)], idx1_i, sem0),
            pltpu.async_copy(rating_h.at[pl.ds(base, bpw)], rat_v, sem0),
            pltpu.async_copy(bias_h, bias_v, sem0),
        ]
        for cp in cps:
            cp.wait()

        for cb in range(nbc):
            def mk_addr(g, _, cb=cb):
                s = pl.multiple_of(g * L, L)
                ru = idx1_u[pl.ds(cb * CHUNK + s, L)]
                ri = idx1_i[pl.ds(cb * CHUNK + s, L)]
                idxr_u[cb, pl.ds(s, L)] = ru
                idxr_i[cb, pl.ds(s, L)] = ri
                for h in range(H):
                    off = h * 1000000
                    addr_u[h * nbc + cb, pl.ds(s, L)] = ru + off
                    addr_i[h * nbc + cb, pl.ds(s, L)] = ri + off
                return 0
            lax.fori_loop(0, CHUNK // L, mk_addr, 0)

        cps = []
        for c in range(nbc):
            cps.append(pltpu.async_copy(ub_h.at[idxr_u.at[c]], bu.at[c], sem0))
            cps.append(pltpu.async_copy(ib_h.at[idxr_i.at[c]], bi.at[c], sem0))

        AHEAD = 8

        def fire(c, _):
            pltpu.async_copy(uw_h.at[addr_u.at[c]], val_u.at[c], sem1)
            pltpu.async_copy(iw_h.at[addr_i.at[c]], val_i.at[c], sem1)
            return 0

        lax.fori_loop(0, AHEAD, fire, 0)

        def drain(c, _):
            pltpu.make_async_copy(uw_h.at[addr_u.at[c]], val_u.at[c],
                                  sem1).wait()
            pltpu.make_async_copy(iw_h.at[addr_i.at[c]], val_i.at[c],
                                  sem1).wait()
            @pl.when(c + AHEAD < ngather)
            def _():
                fire(c + AHEAD, 0)
            return 0
        lax.fori_loop(0, ngather, drain, 0)
        for cp in cps:
            cp.wait()

        bias_bc = plsc.load_gather(bias_v, [jnp.zeros((L,), jnp.int32)])
        loss_vec = jnp.zeros((L,), jnp.float32)
        for cb in range(nbc):
            def group(g, lv, cb=cb):
                s = pl.multiple_of(g * L, L)
                ubv = bu[cb, pl.ds(s, L)]
                ibv = bi[cb, pl.ds(s, L)]
                acc = jnp.zeros((L,), jnp.float32)
                for h in range(H):
                    uv = val_u[h * nbc + cb, pl.ds(s, L)] + ubv
                    iv = val_i[h * nbc + cb, pl.ds(s, L)] + ibv
                    acc = acc + uv * iv
                tgt = acc + bias_bc
                out_v[pl.ds(cb * CHUNK + s, L)] = tgt
                d = tgt - rat_v[pl.ds(cb * CHUNK + s, L)]
                return lv + d * d
            loss_vec = lax.fori_loop(0, CHUNK // L, group, loss_vec)

        loss_v[...] = loss_vec
        pltpu.sync_copy(out_v, out_h.at[pl.ds(base, bpw)])
        pltpu.sync_copy(loss_v, part_h.at[wid])

    return k


def kernel(user, item, rating, user_weight, item_weight, user_bias,
           item_bias, bias):
    B = user.shape[0]
    H = user_weight.shape[1]
    target, partials = _mf_sc(B, H)(
        user, item, rating, user_weight.T.reshape(-1),
        item_weight.T.reshape(-1),
        user_bias.reshape(-1), item_bias.reshape(-1), bias)
    loss = jnp.sum(partials) / B
    return target, loss


# speedup vs baseline: 10.0080x; 10.0080x over previous
import functools

import jax
import jax.numpy as jnp
from jax import lax
from jax.experimental import pallas as pl
from jax.experimental.pallas import tpu as pltpu
from jax.experimental.pallas import tpu_sc as plsc

NC = 2
NS = 16
NW = NC * NS
L = 16
CHUNK = 128

TILE_COLS = 7813
PLANE = TILE_COLS * 1024


KTILE = 128


def _relayout_body(a, b, oa, ob):
    for src, dst in ((a, oa), (b, ob)):
        x = src[...]
        for t in range(KTILE):
            dst[0, t, :, :] = x[:, t * 128:(t + 1) * 128]


def _relayout(wt, it):
    jblocks = (TILE_COLS + KTILE - 1) // KTILE
    return pl.pallas_call(
        _relayout_body,
        grid=(2, jblocks),
        in_specs=[pl.BlockSpec((8, KTILE * 128), lambda i, j: (i, j)),
                  pl.BlockSpec((8, KTILE * 128), lambda i, j: (i, j))],
        out_specs=[
            pl.BlockSpec((1, KTILE, 8, 128), lambda i, j: (i, j, 0, 0)),
            pl.BlockSpec((1, KTILE, 8, 128), lambda i, j: (i, j, 0, 0))],
        out_shape=[jax.ShapeDtypeStruct((2, TILE_COLS, 8, 128), jnp.float32),
                   jax.ShapeDtypeStruct((2, TILE_COLS, 8, 128), jnp.float32)],
    )(wt, it)


def _mf_sc(B, H):
    assert B % (NW * CHUNK) == 0 and H == L
    bpw = B // NW
    nbc = bpw // CHUNK
    ngather = bpw * H // CHUNK

    mesh = plsc.VectorSubcoreMesh(
        core_axis_name="c", subcore_axis_name="s",
        num_cores=NC, num_subcores=NS)

    @functools.partial(
        pl.kernel,
        mesh=mesh,
        compiler_params=pltpu.CompilerParams(
            needs_layout_passes=False, use_tc_tiling_on_sc=False),
        out_type=[
            jax.ShapeDtypeStruct((B,), jnp.float32),
            jax.ShapeDtypeStruct((NW, L), jnp.float32),
        ],
        scratch_types=[
            pltpu.VMEM((bpw,), jnp.int32),
            pltpu.VMEM((bpw,), jnp.int32),
            pltpu.VMEM((nbc, CHUNK), jnp.int32),
            pltpu.VMEM((nbc, CHUNK), jnp.int32),
            pltpu.VMEM((ngather, CHUNK), jnp.int32),
            pltpu.VMEM((ngather, CHUNK), jnp.int32),
            pltpu.VMEM((ngather, CHUNK), jnp.float32),
            pltpu.VMEM((ngather, CHUNK), jnp.float32),
            pltpu.VMEM((nbc, CHUNK), jnp.float32),
            pltpu.VMEM((nbc, CHUNK), jnp.float32),
            pltpu.VMEM((bpw,), jnp.float32),
            pltpu.VMEM((bpw,), jnp.float32),
            pltpu.VMEM((L,), jnp.float32),
            pltpu.VMEM((1,), jnp.float32),
            pltpu.SemaphoreType.DMA,
            pltpu.SemaphoreType.DMA,
        ],
    )
    def k(user_h, item_h, rating_h, uw_h, iw_h, ub_h, ib_h, bias_h,
          out_h, part_h,
          idx1_u, idx1_i, idxr_u, idxr_i,
          addr_u, addr_i, val_u, val_i, bu, bi,
          rat_v, out_v, loss_v, bias_v, sem0, sem1):
        wid = lax.axis_index("s") * NC + lax.axis_index("c")
        base = wid * bpw
        iota = lax.iota(jnp.int32, L)

        cps = [
            pltpu.async_copy(user_h.at[pl.ds(base, bpw)], idx1_u, sem0),
            pltpu.async_copy(item_h.at[pl.ds(base, bpw)], idx1_i, sem0),
            pltpu.async_copy(rating_h.at[pl.ds(base, bpw)], rat_v, sem0),
            pltpu.async_copy(bias_h, bias_v, sem0),
        ]
        for cp in cps:
            cp.wait()

        for cb in range(nbc):
            def mk_addr(g, _, cb=cb):
                s = pl.multiple_of(g * L, L)
                ru = idx1_u[pl.ds(cb * CHUNK + s, L)]
                ri = idx1_i[pl.ds(cb * CHUNK + s, L)]
                idxr_u[cb, pl.ds(s, L)] = ru
                idxr_i[cb, pl.ds(s, L)] = ri
                rpu = (lax.shift_left(lax.shift_right_logical(ru, 7), 10)
                       + jnp.bitwise_and(ru, 127))
                rpi = (lax.shift_left(lax.shift_right_logical(ri, 7), 10)
                       + jnp.bitwise_and(ri, 127))
                for h in range(H):
                    off = (h // 8) * PLANE + (h % 8) * 128
                    addr_u[h * nbc + cb, pl.ds(s, L)] = rpu + off
                    addr_i[h * nbc + cb, pl.ds(s, L)] = rpi + off
                return 0
            lax.fori_loop(0, CHUNK // L, mk_addr, 0)

        cps = []
        for c in range(nbc):
            cps.append(pltpu.async_copy(ub_h.at[idxr_u.at[c]], bu.at[c], sem0))
            cps.append(pltpu.async_copy(ib_h.at[idxr_i.at[c]], bi.at[c], sem0))

        AHEAD = 8

        def fire(c, _):
            pltpu.async_copy(uw_h.at[addr_u.at[c]], val_u.at[c], sem1)
            pltpu.async_copy(iw_h.at[addr_i.at[c]], val_i.at[c], sem1)
            return 0

        lax.fori_loop(0, AHEAD, fire, 0)

        def drain(c, _):
            pltpu.make_async_copy(uw_h.at[addr_u.at[c]], val_u.at[c],
                                  sem1).wait()
            pltpu.make_async_copy(iw_h.at[addr_i.at[c]], val_i.at[c],
                                  sem1).wait()
            @pl.when(c + AHEAD < ngather)
            def _():
                fire(c + AHEAD, 0)
            return 0
        lax.fori_loop(0, ngather, drain, 0)
        for cp in cps:
            cp.wait()

        bias_bc = plsc.load_gather(bias_v, [jnp.zeros((L,), jnp.int32)])
        loss_vec = jnp.zeros((L,), jnp.float32)
        for cb in range(nbc):
            def group(g, lv, cb=cb):
                s = pl.multiple_of(g * L, L)
                ubv = bu[cb, pl.ds(s, L)]
                ibv = bi[cb, pl.ds(s, L)]
                acc = jnp.zeros((L,), jnp.float32)
                for h in range(H):
                    uv = val_u[h * nbc + cb, pl.ds(s, L)] + ubv
                    iv = val_i[h * nbc + cb, pl.ds(s, L)] + ibv
                    acc = acc + uv * iv
                tgt = acc + bias_bc
                out_v[pl.ds(cb * CHUNK + s, L)] = tgt
                d = tgt - rat_v[pl.ds(cb * CHUNK + s, L)]
                return lv + d * d
            loss_vec = lax.fori_loop(0, CHUNK // L, group, loss_vec)

        loss_v[...] = loss_vec
        pltpu.sync_copy(out_v, out_h.at[pl.ds(base, bpw)])
        pltpu.sync_copy(loss_v, part_h.at[wid])

    return k


def kernel(user, item, rating, user_weight, item_weight, user_bias,
           item_bias, bias):
    B = user.shape[0]
    H = user_weight.shape[1]
    uwt, iwt = _relayout(user_weight.T, item_weight.T)
    target, partials = _mf_sc(B, H)(
        user, item, rating, uwt.reshape(-1), iwt.reshape(-1),
        user_bias.reshape(-1), item_bias.reshape(-1), bias)
    loss = jnp.sum(partials) / B
    return target, loss


# speedup vs baseline: 11.7690x; 1.1760x over previous
import functools

import jax
import jax.numpy as jnp
from jax import lax
from jax.experimental import pallas as pl
from jax.experimental.pallas import tpu as pltpu
from jax.experimental.pallas import tpu_sc as plsc

NC = 2
NS = 16
NW = NC * NS
L = 16
CHUNK = 128

TILE_COLS = 7813
PLANE = TILE_COLS * 1024


KTILE = 256


def _relayout_body(a, b, oa, ob):
    for src, dst in ((a, oa), (b, ob)):
        x = src[...]
        for t in range(KTILE):
            dst[0, t, :, :] = x[:, t * 128:(t + 1) * 128]


def _relayout(wt, it):
    jblocks = (TILE_COLS + KTILE - 1) // KTILE
    return pl.pallas_call(
        _relayout_body,
        grid=(2, jblocks),
        in_specs=[pl.BlockSpec((8, KTILE * 128), lambda i, j: (i, j)),
                  pl.BlockSpec((8, KTILE * 128), lambda i, j: (i, j))],
        out_specs=[
            pl.BlockSpec((1, KTILE, 8, 128), lambda i, j: (i, j, 0, 0)),
            pl.BlockSpec((1, KTILE, 8, 128), lambda i, j: (i, j, 0, 0))],
        out_shape=[jax.ShapeDtypeStruct((2, TILE_COLS, 8, 128), jnp.float32),
                   jax.ShapeDtypeStruct((2, TILE_COLS, 8, 128), jnp.float32)],
    )(wt, it)


def _mf_sc(B, H):
    assert B % (NW * CHUNK) == 0 and H == L
    bpw = B // NW
    nbc = bpw // CHUNK
    ngather = bpw * H // CHUNK

    mesh = plsc.VectorSubcoreMesh(
        core_axis_name="c", subcore_axis_name="s",
        num_cores=NC, num_subcores=NS)

    @functools.partial(
        pl.kernel,
        mesh=mesh,
        compiler_params=pltpu.CompilerParams(
            needs_layout_passes=False, use_tc_tiling_on_sc=False),
        out_type=[
            jax.ShapeDtypeStruct((B,), jnp.float32),
            jax.ShapeDtypeStruct((NW, L), jnp.float32),
        ],
        scratch_types=[
            pltpu.VMEM((bpw,), jnp.int32),
            pltpu.VMEM((bpw,), jnp.int32),
            pltpu.VMEM((nbc, CHUNK), jnp.int32),
            pltpu.VMEM((nbc, CHUNK), jnp.int32),
            pltpu.VMEM((ngather, CHUNK), jnp.int32),
            pltpu.VMEM((ngather, CHUNK), jnp.int32),
            pltpu.VMEM((ngather, CHUNK), jnp.float32),
            pltpu.VMEM((ngather, CHUNK), jnp.float32),
            pltpu.VMEM((nbc, CHUNK), jnp.float32),
            pltpu.VMEM((nbc, CHUNK), jnp.float32),
            pltpu.VMEM((bpw,), jnp.float32),
            pltpu.VMEM((bpw,), jnp.float32),
            pltpu.VMEM((L,), jnp.float32),
            pltpu.VMEM((1,), jnp.float32),
            pltpu.SemaphoreType.DMA,
            pltpu.SemaphoreType.DMA,
        ],
    )
    def k(user_h, item_h, rating_h, uw_h, iw_h, ub_h, ib_h, bias_h,
          out_h, part_h,
          idx1_u, idx1_i, idxr_u, idxr_i,
          addr_u, addr_i, val_u, val_i, bu, bi,
          rat_v, out_v, loss_v, bias_v, sem0, sem1):
        wid = lax.axis_index("s") * NC + lax.axis_index("c")
        base = wid * bpw
        iota = lax.iota(jnp.int32, L)

        cps = [
            pltpu.async_copy(user_h.at[pl.ds(base, bpw)], idx1_u, sem0),
            pltpu.async_copy(item_h.at[pl.ds(base, bpw)], idx1_i, sem0),
            pltpu.async_copy(rating_h.at[pl.ds(base, bpw)], rat_v, sem0),
            pltpu.async_copy(bias_h, bias_v, sem0),
        ]
        for cp in cps:
            cp.wait()

        for cb in range(nbc):
            def mk_addr(g, _, cb=cb):
                s = pl.multiple_of(g * L, L)
                ru = idx1_u[pl.ds(cb * CHUNK + s, L)]
                ri = idx1_i[pl.ds(cb * CHUNK + s, L)]
                idxr_u[cb, pl.ds(s, L)] = ru
                idxr_i[cb, pl.ds(s, L)] = ri
                rpu = (lax.shift_left(lax.shift_right_logical(ru, 7), 10)
                       + jnp.bitwise_and(ru, 127))
                rpi = (lax.shift_left(lax.shift_right_logical(ri, 7), 10)
                       + jnp.bitwise_and(ri, 127))
                for h in range(H):
                    off = (h // 8) * PLANE + (h % 8) * 128
                    addr_u[h * nbc + cb, pl.ds(s, L)] = rpu + off
                    addr_i[h * nbc + cb, pl.ds(s, L)] = rpi + off
                return 0
            lax.fori_loop(0, CHUNK // L, mk_addr, 0)

        cps = []
        for c in range(nbc):
            cps.append(pltpu.async_copy(ub_h.at[idxr_u.at[c]], bu.at[c], sem0))
            cps.append(pltpu.async_copy(ib_h.at[idxr_i.at[c]], bi.at[c], sem0))

        AHEAD = 8

        def fire(c, _):
            pltpu.async_copy(uw_h.at[addr_u.at[c]], val_u.at[c], sem1)
            pltpu.async_copy(iw_h.at[addr_i.at[c]], val_i.at[c], sem1)
            return 0

        lax.fori_loop(0, AHEAD, fire, 0)

        def drain(c, _):
            pltpu.make_async_copy(uw_h.at[addr_u.at[c]], val_u.at[c],
                                  sem1).wait()
            pltpu.make_async_copy(iw_h.at[addr_i.at[c]], val_i.at[c],
                                  sem1).wait()
            @pl.when(c + AHEAD < ngather)
            def _():
                fire(c + AHEAD, 0)
            return 0
        lax.fori_loop(0, ngather, drain, 0)
        for cp in cps:
            cp.wait()

        bias_bc = plsc.load_gather(bias_v, [jnp.zeros((L,), jnp.int32)])
        loss_vec = jnp.zeros((L,), jnp.float32)
        for cb in range(nbc):
            def group(g, lv, cb=cb):
                s = pl.multiple_of(g * L, L)
                ubv = bu[cb, pl.ds(s, L)]
                ibv = bi[cb, pl.ds(s, L)]
                acc = jnp.zeros((L,), jnp.float32)
                for h in range(H):
                    uv = val_u[h * nbc + cb, pl.ds(s, L)] + ubv
                    iv = val_i[h * nbc + cb, pl.ds(s, L)] + ibv
                    acc = acc + uv * iv
                tgt = acc + bias_bc
                out_v[pl.ds(cb * CHUNK + s, L)] = tgt
                d = tgt - rat_v[pl.ds(cb * CHUNK + s, L)]
                return lv + d * d
            loss_vec = lax.fori_loop(0, CHUNK // L, group, loss_vec)

        loss_v[...] = loss_vec
        pltpu.sync_copy(out_v, out_h.at[pl.ds(base, bpw)])
        pltpu.sync_copy(loss_v, part_h.at[wid])

    return k


def kernel(user, item, rating, user_weight, item_weight, user_bias,
           item_bias, bias):
    B = user.shape[0]
    H = user_weight.shape[1]
    uwt, iwt = _relayout(user_weight.T, item_weight.T)
    target, partials = _mf_sc(B, H)(
        user, item, rating, uwt.reshape(-1), iwt.reshape(-1),
        user_bias.reshape(-1), item_bias.reshape(-1), bias)
    loss = jnp.sum(partials) / B
    return target, loss


# speedup vs baseline: 12.1356x; 1.0311x over previous
import functools

import jax
import jax.numpy as jnp
from jax import lax
from jax.experimental import pallas as pl
from jax.experimental.pallas import tpu as pltpu
from jax.experimental.pallas import tpu_sc as plsc

NC = 2
NS = 16
NW = NC * NS
L = 16
CHUNK = 128

TILE_COLS = 7813
PLANE = TILE_COLS * 1024


KTILE = 512


def _relayout_body(a, b, oa, ob):
    for src, dst in ((a, oa), (b, ob)):
        x = src[...]
        for t in range(KTILE):
            dst[0, t, :, :] = x[:, t * 128:(t + 1) * 128]


def _relayout(wt, it):
    jblocks = (TILE_COLS + KTILE - 1) // KTILE
    return pl.pallas_call(
        _relayout_body,
        grid=(2, jblocks),
        in_specs=[pl.BlockSpec((8, KTILE * 128), lambda i, j: (i, j)),
                  pl.BlockSpec((8, KTILE * 128), lambda i, j: (i, j))],
        out_specs=[
            pl.BlockSpec((1, KTILE, 8, 128), lambda i, j: (i, j, 0, 0)),
            pl.BlockSpec((1, KTILE, 8, 128), lambda i, j: (i, j, 0, 0))],
        out_shape=[jax.ShapeDtypeStruct((2, TILE_COLS, 8, 128), jnp.float32),
                   jax.ShapeDtypeStruct((2, TILE_COLS, 8, 128), jnp.float32)],
    )(wt, it)


def _mf_sc(B, H):
    assert B % (NW * CHUNK) == 0 and H == L
    bpw = B // NW
    nbc = bpw // CHUNK
    ngather = bpw * H // CHUNK

    mesh = plsc.VectorSubcoreMesh(
        core_axis_name="c", subcore_axis_name="s",
        num_cores=NC, num_subcores=NS)

    @functools.partial(
        pl.kernel,
        mesh=mesh,
        compiler_params=pltpu.CompilerParams(
            needs_layout_passes=False, use_tc_tiling_on_sc=False),
        out_type=[
            jax.ShapeDtypeStruct((B,), jnp.float32),
            jax.ShapeDtypeStruct((NW, L), jnp.float32),
        ],
        scratch_types=[
            pltpu.VMEM((bpw,), jnp.int32),
            pltpu.VMEM((bpw,), jnp.int32),
            pltpu.VMEM((nbc, CHUNK), jnp.int32),
            pltpu.VMEM((nbc, CHUNK), jnp.int32),
            pltpu.VMEM((ngather, CHUNK), jnp.int32),
            pltpu.VMEM((ngather, CHUNK), jnp.int32),
            pltpu.VMEM((ngather, CHUNK), jnp.float32),
            pltpu.VMEM((ngather, CHUNK), jnp.float32),
            pltpu.VMEM((nbc, CHUNK), jnp.float32),
            pltpu.VMEM((nbc, CHUNK), jnp.float32),
            pltpu.VMEM((bpw,), jnp.float32),
            pltpu.VMEM((bpw,), jnp.float32),
            pltpu.VMEM((L,), jnp.float32),
            pltpu.VMEM((1,), jnp.float32),
            pltpu.SemaphoreType.DMA,
            pltpu.SemaphoreType.DMA,
        ],
    )
    def k(user_h, item_h, rating_h, uw_h, iw_h, ub_h, ib_h, bias_h,
          out_h, part_h,
          idx1_u, idx1_i, idxr_u, idxr_i,
          addr_u, addr_i, val_u, val_i, bu, bi,
          rat_v, out_v, loss_v, bias_v, sem0, sem1):
        wid = lax.axis_index("s") * NC + lax.axis_index("c")
        base = wid * bpw
        iota = lax.iota(jnp.int32, L)

        cps = [
            pltpu.async_copy(user_h.at[pl.ds(base, bpw)], idx1_u, sem0),
            pltpu.async_copy(item_h.at[pl.ds(base, bpw)], idx1_i, sem0),
            pltpu.async_copy(rating_h.at[pl.ds(base, bpw)], rat_v, sem0),
            pltpu.async_copy(bias_h, bias_v, sem0),
        ]
        for cp in cps:
            cp.wait()

        for cb in range(nbc):
            def mk_addr(g, _, cb=cb):
                s = pl.multiple_of(g * L, L)
                ru = idx1_u[pl.ds(cb * CHUNK + s, L)]
                ri = idx1_i[pl.ds(cb * CHUNK + s, L)]
                idxr_u[cb, pl.ds(s, L)] = ru
                idxr_i[cb, pl.ds(s, L)] = ri
                rpu = (lax.shift_left(lax.shift_right_logical(ru, 7), 10)
                       + jnp.bitwise_and(ru, 127))
                rpi = (lax.shift_left(lax.shift_right_logical(ri, 7), 10)
                       + jnp.bitwise_and(ri, 127))
                for h in range(H):
                    off = (h // 8) * PLANE + (h % 8) * 128
                    addr_u[h * nbc + cb, pl.ds(s, L)] = rpu + off
                    addr_i[h * nbc + cb, pl.ds(s, L)] = rpi + off
                return 0
            lax.fori_loop(0, CHUNK // L, mk_addr, 0)

        cps = []
        for c in range(nbc):
            cps.append(pltpu.async_copy(ub_h.at[idxr_u.at[c]], bu.at[c], sem0))
            cps.append(pltpu.async_copy(ib_h.at[idxr_i.at[c]], bi.at[c], sem0))

        AHEAD = 8

        def fire(c, _):
            pltpu.async_copy(uw_h.at[addr_u.at[c]], val_u.at[c], sem1)
            pltpu.async_copy(iw_h.at[addr_i.at[c]], val_i.at[c], sem1)
            return 0

        lax.fori_loop(0, AHEAD, fire, 0)

        def drain(c, _):
            pltpu.make_async_copy(uw_h.at[addr_u.at[c]], val_u.at[c],
                                  sem1).wait()
            pltpu.make_async_copy(iw_h.at[addr_i.at[c]], val_i.at[c],
                                  sem1).wait()
            @pl.when(c + AHEAD < ngather)
            def _():
                fire(c + AHEAD, 0)
            return 0
        lax.fori_loop(0, ngather, drain, 0)
        for cp in cps:
            cp.wait()

        bias_bc = plsc.load_gather(bias_v, [jnp.zeros((L,), jnp.int32)])
        loss_vec = jnp.zeros((L,), jnp.float32)
        for cb in range(nbc):
            def group(g, lv, cb=cb):
                s = pl.multiple_of(g * L, L)
                ubv = bu[cb, pl.ds(s, L)]
                ibv = bi[cb, pl.ds(s, L)]
                acc = jnp.zeros((L,), jnp.float32)
                for h in range(H):
                    uv = val_u[h * nbc + cb, pl.ds(s, L)] + ubv
                    iv = val_i[h * nbc + cb, pl.ds(s, L)] + ibv
                    acc = acc + uv * iv
                tgt = acc + bias_bc
                out_v[pl.ds(cb * CHUNK + s, L)] = tgt
                d = tgt - rat_v[pl.ds(cb * CHUNK + s, L)]
                return lv + d * d
            loss_vec = lax.fori_loop(0, CHUNK // L, group, loss_vec)

        loss_v[...] = loss_vec
        pltpu.sync_copy(out_v, out_h.at[pl.ds(base, bpw)])
        pltpu.sync_copy(loss_v, part_h.at[wid])

    return k


def kernel(user, item, rating, user_weight, item_weight, user_bias,
           item_bias, bias):
    B = user.shape[0]
    H = user_weight.shape[1]
    uwt, iwt = _relayout(user_weight.T, item_weight.T)
    target, partials = _mf_sc(B, H)(
        user, item, rating, uwt.reshape(-1), iwt.reshape(-1),
        user_bias.reshape(-1), item_bias.reshape(-1), bias)
    loss = jnp.sum(partials) / B
    return target, loss


# speedup vs baseline: 12.2763x; 1.0116x over previous
import functools

import jax
import jax.numpy as jnp
from jax import lax
from jax.experimental import pallas as pl
from jax.experimental.pallas import tpu as pltpu
from jax.experimental.pallas import tpu_sc as plsc

NC = 2
NS = 16
NW = NC * NS
L = 16
CHUNK = 128

TILE_COLS = 7813
PLANE = TILE_COLS * 1024


KTILE = 1024


def _relayout_body(a, b, oa, ob):
    for src, dst in ((a, oa), (b, ob)):
        x = src[...]
        for t in range(KTILE):
            dst[0, t, :, :] = x[:, t * 128:(t + 1) * 128]


def _relayout(wt, it):
    jblocks = (TILE_COLS + KTILE - 1) // KTILE
    return pl.pallas_call(
        _relayout_body,
        grid=(2, jblocks),
        in_specs=[pl.BlockSpec((8, KTILE * 128), lambda i, j: (i, j)),
                  pl.BlockSpec((8, KTILE * 128), lambda i, j: (i, j))],
        out_specs=[
            pl.BlockSpec((1, KTILE, 8, 128), lambda i, j: (i, j, 0, 0)),
            pl.BlockSpec((1, KTILE, 8, 128), lambda i, j: (i, j, 0, 0))],
        out_shape=[jax.ShapeDtypeStruct((2, TILE_COLS, 8, 128), jnp.float32),
                   jax.ShapeDtypeStruct((2, TILE_COLS, 8, 128), jnp.float32)],
    )(wt, it)


def _mf_sc(B, H):
    assert B % (NW * CHUNK) == 0 and H == L
    bpw = B // NW
    nbc = bpw // CHUNK
    ngather = bpw * H // CHUNK

    mesh = plsc.VectorSubcoreMesh(
        core_axis_name="c", subcore_axis_name="s",
        num_cores=NC, num_subcores=NS)

    @functools.partial(
        pl.kernel,
        mesh=mesh,
        compiler_params=pltpu.CompilerParams(
            needs_layout_passes=False, use_tc_tiling_on_sc=False),
        out_type=[
            jax.ShapeDtypeStruct((B,), jnp.float32),
            jax.ShapeDtypeStruct((NW, L), jnp.float32),
        ],
        scratch_types=[
            pltpu.VMEM((bpw,), jnp.int32),
            pltpu.VMEM((bpw,), jnp.int32),
            pltpu.VMEM((nbc, CHUNK), jnp.int32),
            pltpu.VMEM((nbc, CHUNK), jnp.int32),
            pltpu.VMEM((ngather, CHUNK), jnp.int32),
            pltpu.VMEM((ngather, CHUNK), jnp.int32),
            pltpu.VMEM((ngather, CHUNK), jnp.float32),
            pltpu.VMEM((ngather, CHUNK), jnp.float32),
            pltpu.VMEM((nbc, CHUNK), jnp.float32),
            pltpu.VMEM((nbc, CHUNK), jnp.float32),
            pltpu.VMEM((bpw,), jnp.float32),
            pltpu.VMEM((bpw,), jnp.float32),
            pltpu.VMEM((L,), jnp.float32),
            pltpu.VMEM((1,), jnp.float32),
            pltpu.SemaphoreType.DMA,
            pltpu.SemaphoreType.DMA,
        ],
    )
    def k(user_h, item_h, rating_h, uw_h, iw_h, ub_h, ib_h, bias_h,
          out_h, part_h,
          idx1_u, idx1_i, idxr_u, idxr_i,
          addr_u, addr_i, val_u, val_i, bu, bi,
          rat_v, out_v, loss_v, bias_v, sem0, sem1):
        wid = lax.axis_index("s") * NC + lax.axis_index("c")
        base = wid * bpw
        iota = lax.iota(jnp.int32, L)

        cps = [
            pltpu.async_copy(user_h.at[pl.ds(base, bpw)], idx1_u, sem0),
            pltpu.async_copy(item_h.at[pl.ds(base, bpw)], idx1_i, sem0),
            pltpu.async_copy(rating_h.at[pl.ds(base, bpw)], rat_v, sem0),
            pltpu.async_copy(bias_h, bias_v, sem0),
        ]
        for cp in cps:
            cp.wait()

        for cb in range(nbc):
            def mk_addr(g, _, cb=cb):
                s = pl.multiple_of(g * L, L)
                ru = idx1_u[pl.ds(cb * CHUNK + s, L)]
                ri = idx1_i[pl.ds(cb * CHUNK + s, L)]
                idxr_u[cb, pl.ds(s, L)] = ru
                idxr_i[cb, pl.ds(s, L)] = ri
                rpu = (lax.shift_left(lax.shift_right_logical(ru, 7), 10)
                       + jnp.bitwise_and(ru, 127))
                rpi = (lax.shift_left(lax.shift_right_logical(ri, 7), 10)
                       + jnp.bitwise_and(ri, 127))
                for h in range(H):
                    off = (h // 8) * PLANE + (h % 8) * 128
                    addr_u[h * nbc + cb, pl.ds(s, L)] = rpu + off
                    addr_i[h * nbc + cb, pl.ds(s, L)] = rpi + off
                return 0
            lax.fori_loop(0, CHUNK // L, mk_addr, 0)

        cps = []
        for c in range(nbc):
            cps.append(pltpu.async_copy(ub_h.at[idxr_u.at[c]], bu.at[c], sem0))
            cps.append(pltpu.async_copy(ib_h.at[idxr_i.at[c]], bi.at[c], sem0))

        AHEAD = 8

        def fire(c, _):
            pltpu.async_copy(uw_h.at[addr_u.at[c]], val_u.at[c], sem1)
            pltpu.async_copy(iw_h.at[addr_i.at[c]], val_i.at[c], sem1)
            return 0

        lax.fori_loop(0, AHEAD, fire, 0)

        def drain(c, _):
            pltpu.make_async_copy(uw_h.at[addr_u.at[c]], val_u.at[c],
                                  sem1).wait()
            pltpu.make_async_copy(iw_h.at[addr_i.at[c]], val_i.at[c],
                                  sem1).wait()
            @pl.when(c + AHEAD < ngather)
            def _():
                fire(c + AHEAD, 0)
            return 0
        lax.fori_loop(0, ngather, drain, 0)
        for cp in cps:
            cp.wait()

        bias_bc = plsc.load_gather(bias_v, [jnp.zeros((L,), jnp.int32)])
        loss_vec = jnp.zeros((L,), jnp.float32)
        for cb in range(nbc):
            def group(g, lv, cb=cb):
                s = pl.multiple_of(g * L, L)
                ubv = bu[cb, pl.ds(s, L)]
                ibv = bi[cb, pl.ds(s, L)]
                acc = jnp.zeros((L,), jnp.float32)
                for h in range(H):
                    uv = val_u[h * nbc + cb, pl.ds(s, L)] + ubv
                    iv = val_i[h * nbc + cb, pl.ds(s, L)] + ibv
                    acc = acc + uv * iv
                tgt = acc + bias_bc
                out_v[pl.ds(cb * CHUNK + s, L)] = tgt
                d = tgt - rat_v[pl.ds(cb * CHUNK + s, L)]
                return lv + d * d
            loss_vec = lax.fori_loop(0, CHUNK // L, group, loss_vec)

        loss_v[...] = loss_vec
        pltpu.sync_copy(out_v, out_h.at[pl.ds(base, bpw)])
        pltpu.sync_copy(loss_v, part_h.at[wid])

    return k


def kernel(user, item, rating, user_weight, item_weight, user_bias,
           item_bias, bias):
    B = user.shape[0]
    H = user_weight.shape[1]
    uwt, iwt = _relayout(user_weight.T, item_weight.T)
    target, partials = _mf_sc(B, H)(
        user, item, rating, uwt.reshape(-1), iwt.reshape(-1),
        user_bias.reshape(-1), item_bias.reshape(-1), bias)
    loss = jnp.sum(partials) / B
    return target, loss


# speedup vs baseline: 16.4821x; 1.3426x over previous
import functools

import jax
import jax.numpy as jnp
from jax import lax
from jax.experimental import pallas as pl
from jax.experimental.pallas import tpu as pltpu
from jax.experimental.pallas import tpu_sc as plsc

NC = 2
NS = 16
NW = NC * NS
L = 16
CHUNK = 128

U = 1000000
CRANGE = 31232
CCHUNK = 6144
TAIL0 = CRANGE * NW
TAIL1 = TAIL0 + 512

mesh = plsc.VectorSubcoreMesh(
    core_axis_name="c", subcore_axis_name="s",
    num_cores=NC, num_subcores=NS)


@functools.partial(
    pl.kernel,
    mesh=mesh,
    compiler_params=pltpu.CompilerParams(
        needs_layout_passes=False, use_tc_tiling_on_sc=True),
    out_type=[
        jax.ShapeDtypeStruct((U * 16,), jnp.float32),
        jax.ShapeDtypeStruct((U * 16,), jnp.float32),
    ],
    scratch_types=(
        [pltpu.VMEM((CCHUNK,), jnp.float32) for _ in range(16)]
        + [pltpu.VMEM((1024,), jnp.float32),
           pltpu.SemaphoreType.DMA,
           pltpu.SemaphoreType.DMA,
           pltpu.SemaphoreType.DMA]
    ),
)
def _copy_k(uwt_h, iwt_h, ut_tail_h, it_tail_h, ud_h, id_h, *scr):
    bufs = (scr[0:8], scr[8:16])
    tbuf = scr[16]
    semr, semw0, semw1 = scr[17:20]
    wsems = (semw0, semw1)
    wid = lax.axis_index("s") * NC + lax.axis_index("c")
    c0 = wid * CRANGE
    pending = {}

    chunks = []
    for src, dst in ((uwt_h, ud_h), (iwt_h, id_h)):
        for i in range(2):
            off = 0
            while off < CRANGE:
                ln = min(CCHUNK, CRANGE - off)
                chunks.append((src, dst, i, off, ln, False))
                off += ln
    for src, dst in ((uwt_h, ud_h), (iwt_h, id_h)):
        for i in range(2):
            chunks.append((src, dst, i, TAIL0, 512, True))

    for kk, (src, dst, i, off, ln, is_tail) in enumerate(chunks):
        b = kk % 2
        if b in pending:
            for w in pending.pop(b):
                w.wait()
        col = (off if is_tail else c0 + off)

        def run(src=src, dst=dst, i=i, col=col, ln=ln, b=b):
            rds = [
                pltpu.async_copy(src.at[8 * i + s, pl.ds(col, ln)],
                                 bufs[b][s].at[pl.ds(0, ln)], semr)
                for s in range(8)
            ]
            for rd in rds:
                rd.wait()
            return [
                pltpu.async_copy(bufs[b][s].at[pl.ds(0, ln)],
                                 dst.at[pl.ds((8 * i + s) * U + col, ln)],
                                 wsems[b])
                for s in range(8)
            ]

        if is_tail:
            @pl.when(wid == 0)
            def _():
                for w in run():
                    w.wait()
        else:
            pending[b] = run()

    for ws in pending.values():
        for w in ws:
            w.wait()

    @pl.when(wid == 0)
    def _():
        for tail, dst in ((ut_tail_h, ud_h), (it_tail_h, id_h)):
            pltpu.async_copy(tail, tbuf, semr).wait()
            ws = [
                pltpu.async_copy(tbuf.at[pl.ds(row * 64, 64)],
                                 dst.at[pl.ds(row * U + TAIL1, 64)], semw0)
                for row in range(16)
            ]
            for w in ws:
                w.wait()


def _mf_sc(B, H):
    assert B % (NW * CHUNK) == 0 and H == L
    bpw = B // NW
    nbc = bpw // CHUNK
    ngather = bpw * H // CHUNK

    @functools.partial(
        pl.kernel,
        mesh=mesh,
        compiler_params=pltpu.CompilerParams(
            needs_layout_passes=False, use_tc_tiling_on_sc=False),
        out_type=[
            jax.ShapeDtypeStruct((B,), jnp.float32),
            jax.ShapeDtypeStruct((NW, L), jnp.float32),
        ],
        scratch_types=[
            pltpu.VMEM((bpw,), jnp.int32),
            pltpu.VMEM((bpw,), jnp.int32),
            pltpu.VMEM((nbc, CHUNK), jnp.int32),
            pltpu.VMEM((nbc, CHUNK), jnp.int32),
            pltpu.VMEM((ngather, CHUNK), jnp.int32),
            pltpu.VMEM((ngather, CHUNK), jnp.int32),
            pltpu.VMEM((ngather, CHUNK), jnp.float32),
            pltpu.VMEM((ngather, CHUNK), jnp.float32),
            pltpu.VMEM((nbc, CHUNK), jnp.float32),
            pltpu.VMEM((nbc, CHUNK), jnp.float32),
            pltpu.VMEM((bpw,), jnp.float32),
            pltpu.VMEM((bpw,), jnp.float32),
            pltpu.VMEM((L,), jnp.float32),
            pltpu.VMEM((1,), jnp.float32),
            pltpu.SemaphoreType.DMA,
            pltpu.SemaphoreType.DMA,
        ],
    )
    def k(user_h, item_h, rating_h, uw_h, iw_h, ub_h, ib_h, bias_h,
          out_h, part_h,
          idx1_u, idx1_i, idxr_u, idxr_i,
          addr_u, addr_i, val_u, val_i, bu, bi,
          rat_v, out_v, loss_v, bias_v, sem0, sem1):
        wid = lax.axis_index("s") * NC + lax.axis_index("c")
        base = wid * bpw

        cps = [
            pltpu.async_copy(user_h.at[pl.ds(base, bpw)], idx1_u, sem0),
            pltpu.async_copy(item_h.at[pl.ds(base, bpw)], idx1_i, sem0),
            pltpu.async_copy(rating_h.at[pl.ds(base, bpw)], rat_v, sem0),
            pltpu.async_copy(bias_h, bias_v, sem0),
        ]
        for cp in cps:
            cp.wait()

        for cb in range(nbc):
            def mk_addr(g, _, cb=cb):
                s = pl.multiple_of(g * L, L)
                ru = idx1_u[pl.ds(cb * CHUNK + s, L)]
                ri = idx1_i[pl.ds(cb * CHUNK + s, L)]
                idxr_u[cb, pl.ds(s, L)] = ru
                idxr_i[cb, pl.ds(s, L)] = ri
                for h in range(H):
                    addr_u[h * nbc + cb, pl.ds(s, L)] = ru + h * U
                    addr_i[h * nbc + cb, pl.ds(s, L)] = ri + h * U
                return 0
            lax.fori_loop(0, CHUNK // L, mk_addr, 0)

        cps = []
        for c in range(nbc):
            cps.append(pltpu.async_copy(ub_h.at[idxr_u.at[c]], bu.at[c], sem0))
            cps.append(pltpu.async_copy(ib_h.at[idxr_i.at[c]], bi.at[c], sem0))

        AHEAD = 8

        def fire(c, _):
            pltpu.async_copy(uw_h.at[addr_u.at[c]], val_u.at[c], sem1)
            pltpu.async_copy(iw_h.at[addr_i.at[c]], val_i.at[c], sem1)
            return 0

        lax.fori_loop(0, AHEAD, fire, 0)

        def drain(c, _):
            pltpu.make_async_copy(uw_h.at[addr_u.at[c]], val_u.at[c],
                                  sem1).wait()
            pltpu.make_async_copy(iw_h.at[addr_i.at[c]], val_i.at[c],
                                  sem1).wait()
            @pl.when(c + AHEAD < ngather)
            def _():
                fire(c + AHEAD, 0)
            return 0
        lax.fori_loop(0, ngather, drain, 0)
        for cp in cps:
            cp.wait()

        bias_bc = plsc.load_gather(bias_v, [jnp.zeros((L,), jnp.int32)])
        loss_vec = jnp.zeros((L,), jnp.float32)
        for cb in range(nbc):
            def group(g, lv, cb=cb):
                s = pl.multiple_of(g * L, L)
                ubv = bu[cb, pl.ds(s, L)]
                ibv = bi[cb, pl.ds(s, L)]
                acc = jnp.zeros((L,), jnp.float32)
                for h in range(H):
                    uv = val_u[h * nbc + cb, pl.ds(s, L)] + ubv
                    iv = val_i[h * nbc + cb, pl.ds(s, L)] + ibv
                    acc = acc + uv * iv
                tgt = acc + bias_bc
                out_v[pl.ds(cb * CHUNK + s, L)] = tgt
                d = tgt - rat_v[pl.ds(cb * CHUNK + s, L)]
                return lv + d * d
            loss_vec = lax.fori_loop(0, CHUNK // L, group, loss_vec)

        loss_v[...] = loss_vec
        pltpu.sync_copy(out_v, out_h.at[pl.ds(base, bpw)])
        pltpu.sync_copy(loss_v, part_h.at[wid])

    return k


def kernel(user, item, rating, user_weight, item_weight, user_bias,
           item_bias, bias):
    B = user.shape[0]
    H = user_weight.shape[1]
    ud, id_ = _copy_k(user_weight.T, item_weight.T,
                      user_weight[TAIL1:].T.reshape(-1),
                      item_weight[TAIL1:].T.reshape(-1))
    target, partials = _mf_sc(B, H)(
        user, item, rating, ud, id_,
        user_bias.reshape(-1), item_bias.reshape(-1), bias)
    loss = jnp.sum(partials) / B
    return target, loss


# speedup vs baseline: 16.6278x; 1.0088x over previous
import functools

import jax
import jax.numpy as jnp
from jax import lax
from jax.experimental import pallas as pl
from jax.experimental.pallas import tpu as pltpu
from jax.experimental.pallas import tpu_sc as plsc

NC = 2
NS = 16
NW = NC * NS
L = 16
CHUNK = 128

U = 1000000
CRANGE = 31232
CCHUNK = 7808
TAIL0 = CRANGE * NW
TAIL1 = TAIL0 + 512

mesh = plsc.VectorSubcoreMesh(
    core_axis_name="c", subcore_axis_name="s",
    num_cores=NC, num_subcores=NS)


@functools.partial(
    pl.kernel,
    mesh=mesh,
    compiler_params=pltpu.CompilerParams(
        needs_layout_passes=False, use_tc_tiling_on_sc=True),
    out_type=[
        jax.ShapeDtypeStruct((U * 16,), jnp.float32),
        jax.ShapeDtypeStruct((U * 16,), jnp.float32),
    ],
    scratch_types=(
        [pltpu.VMEM((CCHUNK,), jnp.float32) for _ in range(16)]
        + [pltpu.VMEM((1024,), jnp.float32),
           pltpu.SemaphoreType.DMA,
           pltpu.SemaphoreType.DMA,
           pltpu.SemaphoreType.DMA]
    ),
)
def _copy_k(uwt_h, iwt_h, ut_tail_h, it_tail_h, ud_h, id_h, *scr):
    bufs = (scr[0:8], scr[8:16])
    tbuf = scr[16]
    semr, semw0, semw1 = scr[17:20]
    wsems = (semw0, semw1)
    wid = lax.axis_index("s") * NC + lax.axis_index("c")
    c0 = wid * CRANGE
    pending = {}

    chunks = []
    for src, dst in ((uwt_h, ud_h), (iwt_h, id_h)):
        for i in range(2):
            off = 0
            while off < CRANGE:
                ln = min(CCHUNK, CRANGE - off)
                chunks.append((src, dst, i, off, ln, False))
                off += ln
    for src, dst in ((uwt_h, ud_h), (iwt_h, id_h)):
        for i in range(2):
            chunks.append((src, dst, i, TAIL0, 512, True))

    for kk, (src, dst, i, off, ln, is_tail) in enumerate(chunks):
        b = kk % 2
        if b in pending:
            for w in pending.pop(b):
                w.wait()
        col = (off if is_tail else c0 + off)

        def run(src=src, dst=dst, i=i, col=col, ln=ln, b=b):
            rds = [
                pltpu.async_copy(src.at[8 * i + s, pl.ds(col, ln)],
                                 bufs[b][s].at[pl.ds(0, ln)], semr)
                for s in range(8)
            ]
            for rd in rds:
                rd.wait()
            return [
                pltpu.async_copy(bufs[b][s].at[pl.ds(0, ln)],
                                 dst.at[pl.ds((8 * i + s) * U + col, ln)],
                                 wsems[b])
                for s in range(8)
            ]

        if is_tail:
            @pl.when(wid == 0)
            def _():
                for w in run():
                    w.wait()
        else:
            pending[b] = run()

    for ws in pending.values():
        for w in ws:
            w.wait()

    @pl.when(wid == 0)
    def _():
        for tail, dst in ((ut_tail_h, ud_h), (it_tail_h, id_h)):
            pltpu.async_copy(tail, tbuf, semr).wait()
            ws = [
                pltpu.async_copy(tbuf.at[pl.ds(row * 64, 64)],
                                 dst.at[pl.ds(row * U + TAIL1, 64)], semw0)
                for row in range(16)
            ]
            for w in ws:
                w.wait()


def _mf_sc(B, H):
    assert B % (NW * CHUNK) == 0 and H == L
    bpw = B // NW
    nbc = bpw // CHUNK
    ngather = bpw * H // CHUNK

    @functools.partial(
        pl.kernel,
        mesh=mesh,
        compiler_params=pltpu.CompilerParams(
            needs_layout_passes=False, use_tc_tiling_on_sc=False),
        out_type=[
            jax.ShapeDtypeStruct((B,), jnp.float32),
            jax.ShapeDtypeStruct((NW, L), jnp.float32),
        ],
        scratch_types=[
            pltpu.VMEM((bpw,), jnp.int32),
            pltpu.VMEM((bpw,), jnp.int32),
            pltpu.VMEM((nbc, CHUNK), jnp.int32),
            pltpu.VMEM((nbc, CHUNK), jnp.int32),
            pltpu.VMEM((ngather, CHUNK), jnp.int32),
            pltpu.VMEM((ngather, CHUNK), jnp.int32),
            pltpu.VMEM((ngather, CHUNK), jnp.float32),
            pltpu.VMEM((ngather, CHUNK), jnp.float32),
            pltpu.VMEM((nbc, CHUNK), jnp.float32),
            pltpu.VMEM((nbc, CHUNK), jnp.float32),
            pltpu.VMEM((bpw,), jnp.float32),
            pltpu.VMEM((bpw,), jnp.float32),
            pltpu.VMEM((L,), jnp.float32),
            pltpu.VMEM((1,), jnp.float32),
            pltpu.SemaphoreType.DMA,
            pltpu.SemaphoreType.DMA,
        ],
    )
    def k(user_h, item_h, rating_h, uw_h, iw_h, ub_h, ib_h, bias_h,
          out_h, part_h,
          idx1_u, idx1_i, idxr_u, idxr_i,
          addr_u, addr_i, val_u, val_i, bu, bi,
          rat_v, out_v, loss_v, bias_v, sem0, sem1):
        wid = lax.axis_index("s") * NC + lax.axis_index("c")
        base = wid * bpw

        cps = [
            pltpu.async_copy(user_h.at[pl.ds(base, bpw)], idx1_u, sem0),
            pltpu.async_copy(item_h.at[pl.ds(base, bpw)], idx1_i, sem0),
            pltpu.async_copy(rating_h.at[pl.ds(base, bpw)], rat_v, sem0),
            pltpu.async_copy(bias_h, bias_v, sem0),
        ]
        for cp in cps:
            cp.wait()

        for cb in range(nbc):
            def mk_addr(g, _, cb=cb):
                s = pl.multiple_of(g * L, L)
                ru = idx1_u[pl.ds(cb * CHUNK + s, L)]
                ri = idx1_i[pl.ds(cb * CHUNK + s, L)]
                idxr_u[cb, pl.ds(s, L)] = ru
                idxr_i[cb, pl.ds(s, L)] = ri
                for h in range(H):
                    addr_u[h * nbc + cb, pl.ds(s, L)] = ru + h * U
                    addr_i[h * nbc + cb, pl.ds(s, L)] = ri + h * U
                return 0
            lax.fori_loop(0, CHUNK // L, mk_addr, 0)

        cps = []
        for c in range(nbc):
            cps.append(pltpu.async_copy(ub_h.at[idxr_u.at[c]], bu.at[c], sem0))
            cps.append(pltpu.async_copy(ib_h.at[idxr_i.at[c]], bi.at[c], sem0))

        AHEAD = 8

        def fire(c, _):
            pltpu.async_copy(uw_h.at[addr_u.at[c]], val_u.at[c], sem1)
            pltpu.async_copy(iw_h.at[addr_i.at[c]], val_i.at[c], sem1)
            return 0

        lax.fori_loop(0, AHEAD, fire, 0)

        def drain(c, _):
            pltpu.make_async_copy(uw_h.at[addr_u.at[c]], val_u.at[c],
                                  sem1).wait()
            pltpu.make_async_copy(iw_h.at[addr_i.at[c]], val_i.at[c],
                                  sem1).wait()
            @pl.when(c + AHEAD < ngather)
            def _():
                fire(c + AHEAD, 0)
            return 0
        lax.fori_loop(0, ngather, drain, 0)
        for cp in cps:
            cp.wait()

        bias_bc = plsc.load_gather(bias_v, [jnp.zeros((L,), jnp.int32)])
        loss_vec = jnp.zeros((L,), jnp.float32)
        for cb in range(nbc):
            def group(g, lv, cb=cb):
                s = pl.multiple_of(g * L, L)
                ubv = bu[cb, pl.ds(s, L)]
                ibv = bi[cb, pl.ds(s, L)]
                acc = jnp.zeros((L,), jnp.float32)
                for h in range(H):
                    uv = val_u[h * nbc + cb, pl.ds(s, L)] + ubv
                    iv = val_i[h * nbc + cb, pl.ds(s, L)] + ibv
                    acc = acc + uv * iv
                tgt = acc + bias_bc
                out_v[pl.ds(cb * CHUNK + s, L)] = tgt
                d = tgt - rat_v[pl.ds(cb * CHUNK + s, L)]
                return lv + d * d
            loss_vec = lax.fori_loop(0, CHUNK // L, group, loss_vec)

        loss_v[...] = loss_vec
        pltpu.sync_copy(out_v, out_h.at[pl.ds(base, bpw)])
        pltpu.sync_copy(loss_v, part_h.at[wid])

    return k


def kernel(user, item, rating, user_weight, item_weight, user_bias,
           item_bias, bias):
    B = user.shape[0]
    H = user_weight.shape[1]
    ud, id_ = _copy_k(user_weight.T, item_weight.T,
                      user_weight[TAIL1:].T.reshape(-1),
                      item_weight[TAIL1:].T.reshape(-1))
    target, partials = _mf_sc(B, H)(
        user, item, rating, ud, id_,
        user_bias.reshape(-1), item_bias.reshape(-1), bias)
    loss = jnp.sum(partials) / B
    return target, loss
